# Initial kernel scaffold; baseline (speedup 1.0000x reference)
import jax
import jax.numpy as jnp
from jax.experimental import pallas as pl


def kernel(x, Wr, br, W1, b1, W2, b2):
    raise NotImplementedError("write your pallas kernel here")


# speedup vs baseline: 1.9341x; 1.9341x over previous
import functools

import jax
import jax.numpy as jnp
from jax import lax
from jax.experimental import pallas as pl
from jax.experimental.pallas import tpu as pltpu
from jax.experimental.pallas import tpu_sc as plsc

D_MODEL = 2048
D_HIDDEN = 8192
N_EXPERTS = 8
N_TOKENS = 4096
CAP = 641
CAPP = 648
DROP_SLOT = CAP
WLANES = 128
D_AUG = D_MODEL + WLANES

TB = 256
NB = N_TOKENS // TB
HB = 512
NHB = D_HIDDEN // HB

NW = 32
TPW = N_TOKENS // NW
SUB = 32
NSUB = TPW // SUB


def _router_kernel(h_ref, wr_ref, br_ref, haug_ref, slot_ref, aux_ref,
                   carry_ref, imp_ref, load_ref):
    b = pl.program_id(0)

    @pl.when(b == 0)
    def _init():
        carry_ref[...] = jnp.zeros_like(carry_ref)
        imp_ref[...] = jnp.zeros_like(imp_ref)
        load_ref[...] = jnp.zeros_like(load_ref)

    h = h_ref[...]
    logits = lax.dot_general(
        h, wr_ref[...], (((1,), (1,)), ((), ())),
        preferred_element_type=jnp.float32) + br_ref[...]

    m = jnp.max(logits, axis=1, keepdims=True)
    ex = jnp.exp(logits - m)
    probs = ex / jnp.sum(ex, axis=1, keepdims=True)

    iota_e = lax.broadcasted_iota(jnp.int32, (TB, N_EXPERTS), 1)
    is_max = logits == m
    top1 = jnp.min(jnp.where(is_max, iota_e, N_EXPERTS),
                   axis=1, keepdims=True)
    assign = (iota_e == top1).astype(jnp.float32)

    r_io = lax.broadcasted_iota(jnp.int32, (TB, TB), 0)
    c_io = lax.broadcasted_iota(jnp.int32, (TB, TB), 1)
    tri = (r_io >= c_io).astype(jnp.bfloat16)
    pos = lax.dot_general(
        tri, assign.astype(jnp.bfloat16), (((1,), (0,)), ((), ())),
        preferred_element_type=jnp.float32) + carry_ref[...]
    pos_tok = jnp.sum(assign * pos, axis=1, keepdims=True) - 1.0
    keep = pos_tok < float(CAP)

    w_tok = jnp.sum(probs * assign, axis=1, keepdims=True)
    w_eff = jnp.where(keep, w_tok, 0.0)
    slot = jnp.where(keep,
                     top1 * CAPP + pos_tok.astype(jnp.int32),
                     DROP_SLOT)

    haug_ref[:, 0:D_MODEL] = h
    haug_ref[:, D_MODEL:D_AUG] = jnp.broadcast_to(w_eff, (TB, WLANES))
    slot_ref[...] = slot

    carry_ref[...] = carry_ref[...] + jnp.sum(assign, axis=0, keepdims=True)
    imp_ref[...] = imp_ref[...] + jnp.sum(probs, axis=0, keepdims=True)
    load_ref[...] = load_ref[...] + jnp.sum(assign, axis=0, keepdims=True)

    @pl.when(b == NB - 1)
    def _finish():
        imp = imp_ref[...] / float(N_TOKENS)
        load = load_ref[...] / float(N_TOKENS)
        lb = float(N_EXPERTS) * jnp.sum(imp * load)
        aux_ref[...] = jnp.full((1, 1), 0.01, jnp.float32) * lb


def _router(h, wr, br2d):
    return pl.pallas_call(
        _router_kernel,
        grid=(NB,),
        in_specs=[
            pl.BlockSpec((TB, D_MODEL), lambda b: (b, 0)),
            pl.BlockSpec((N_EXPERTS, D_MODEL), lambda b: (0, 0)),
            pl.BlockSpec((1, N_EXPERTS), lambda b: (0, 0)),
        ],
        out_specs=[
            pl.BlockSpec((TB, D_AUG), lambda b: (b, 0)),
            pl.BlockSpec((TB, 1), lambda b: (b, 0)),
            pl.BlockSpec((1, 1), lambda b: (0, 0)),
        ],
        out_shape=[
            jax.ShapeDtypeStruct((N_TOKENS, D_AUG), jnp.float32),
            jax.ShapeDtypeStruct((N_TOKENS, 1), jnp.int32),
            jax.ShapeDtypeStruct((1, 1), jnp.float32),
        ],
        scratch_shapes=[
            pltpu.VMEM((1, N_EXPERTS), jnp.float32),
            pltpu.VMEM((1, N_EXPERTS), jnp.float32),
            pltpu.VMEM((1, N_EXPERTS), jnp.float32),
        ],
    )(h, wr, br2d)


def _dispatch_sc(h_aug, slot3d):
    mesh = plsc.VectorSubcoreMesh(core_axis_name="c", subcore_axis_name="s")

    @functools.partial(
        pl.kernel, mesh=mesh,
        out_type=jax.ShapeDtypeStruct((N_EXPERTS * CAPP, D_AUG), jnp.float32),
        scratch_types=[
            pltpu.VMEM((NSUB, SUB), jnp.int32),
            pltpu.VMEM((SUB, D_AUG), jnp.float32),
            pltpu.SemaphoreType.DMA,
        ],
    )
    def k(haug_hbm, slot_hbm, xe_hbm, idx_v, rows_v, sem):
        wid = lax.axis_index("s") * 2 + lax.axis_index("c")
        pltpu.sync_copy(slot_hbm.at[wid], idx_v)
        for j in range(NSUB):
            base = wid * TPW + j * SUB
            pltpu.sync_copy(haug_hbm.at[pl.ds(base, SUB)], rows_v)
            pltpu.async_copy(rows_v, xe_hbm.at[idx_v.at[j]], sem).wait()

    return k(h_aug, slot3d)


def _mlp_kernel(x_ref, wcol_ref, w1_ref, b1_ref, w2_ref, b2_ref, y_ref,
                acc_ref, xbf_ref):
    hb = pl.program_id(1)

    @pl.when(hb == 0)
    def _start():
        xbf_ref[...] = x_ref[...].astype(jnp.bfloat16)
        acc_ref[...] = jnp.zeros_like(acc_ref)

    w1b = w1_ref[0].astype(jnp.bfloat16)
    he = lax.dot_general(
        xbf_ref[...], w1b, (((1,), (1,)), ((), ())),
        preferred_element_type=jnp.float32)
    he = jax.nn.gelu(he + b1_ref[0])
    w2b = w2_ref[0].astype(jnp.bfloat16)
    acc_ref[...] += lax.dot_general(
        he.astype(jnp.bfloat16), w2b, (((1,), (1,)), ((), ())),
        preferred_element_type=jnp.float32)

    @pl.when(hb == NHB - 1)
    def _finish():
        wcol = wcol_ref[:, 0:1]
        y_ref[...] = (acc_ref[...] + b2_ref[0]) * wcol


def _expert_mlp(xe_all, w1, b1, w2, b2):
    return pl.pallas_call(
        _mlp_kernel,
        grid=(N_EXPERTS, NHB),
        in_specs=[
            pl.BlockSpec((CAPP, D_MODEL), lambda e, hb: (e, 0)),
            pl.BlockSpec((CAPP, WLANES), lambda e, hb: (e, D_MODEL // WLANES)),
            pl.BlockSpec((1, HB, D_MODEL), lambda e, hb: (e, hb, 0)),
            pl.BlockSpec((1, 1, HB), lambda e, hb: (e * NHB + hb, 0, 0)),
            pl.BlockSpec((1, D_MODEL, HB), lambda e, hb: (e, 0, hb)),
            pl.BlockSpec((1, 1, D_MODEL), lambda e, hb: (e, 0, 0)),
        ],
        out_specs=pl.BlockSpec((CAPP, D_MODEL), lambda e, hb: (e, 0)),
        out_shape=jax.ShapeDtypeStruct((N_EXPERTS * CAPP, D_MODEL),
                                       jnp.float32),
        scratch_shapes=[
            pltpu.VMEM((CAPP, D_MODEL), jnp.float32),
            pltpu.VMEM((CAPP, D_MODEL), jnp.bfloat16),
        ],
        compiler_params=pltpu.CompilerParams(
            dimension_semantics=("arbitrary", "arbitrary"),
            vmem_limit_bytes=60 * 1024 * 1024,
        ),
    )(xe_all, xe_all, w1, b1.reshape(N_EXPERTS * NHB, 1, HB),
      w2, b2.reshape(N_EXPERTS, 1, D_MODEL))


def _combine_sc(ye_all, slot3d):
    mesh = plsc.VectorSubcoreMesh(core_axis_name="c", subcore_axis_name="s")

    @functools.partial(
        pl.kernel, mesh=mesh,
        out_type=jax.ShapeDtypeStruct((N_TOKENS, D_MODEL), jnp.float32),
        scratch_types=[
            pltpu.VMEM((NSUB, SUB), jnp.int32),
            pltpu.VMEM((SUB, D_MODEL), jnp.float32),
            pltpu.SemaphoreType.DMA,
        ],
    )
    def k(ye_hbm, slot_hbm, out_hbm, idx_v, rows_v, sem):
        wid = lax.axis_index("s") * 2 + lax.axis_index("c")
        pltpu.sync_copy(slot_hbm.at[wid], idx_v)
        for j in range(NSUB):
            base = wid * TPW + j * SUB
            pltpu.async_copy(ye_hbm.at[idx_v.at[j]], rows_v, sem).wait()
            pltpu.sync_copy(rows_v, out_hbm.at[pl.ds(base, SUB)])

    return k(ye_all, slot3d)


def kernel(x, Wr, br, W1, b1, W2, b2):
    Bv, Tv, Dv = x.shape
    h = x.reshape(Bv * Tv, Dv)
    h_aug, slot, aux = _router(h, Wr, br.reshape(1, N_EXPERTS))
    slot3d = slot.reshape(NW, NSUB, SUB)
    xe_all = _dispatch_sc(h_aug, slot3d)
    ye_all = _expert_mlp(xe_all, W1, b1, W2, b2)
    out = _combine_sc(ye_all, slot3d)
    return out.reshape(Bv, Tv, Dv), aux.reshape(())
